# Initial kernel scaffold; baseline (speedup 1.0000x reference)
import jax
import jax.numpy as jnp
from jax.experimental import pallas as pl


def kernel(x, edge_index, W_self1, W_neigh1, b1, W_self2, W_neigh2, b2):
    raise NotImplementedError("write your pallas kernel here")


# speedup vs baseline: 3.9672x; 3.9672x over previous
import functools

import jax
import jax.numpy as jnp
from jax import lax
from jax.experimental import pallas as pl
from jax.experimental.pallas import tpu as pltpu
from jax.experimental.pallas import tpu_sc as plsc

N = 10000
E = 320000
D = 128
NC = 2
NS = 16
NW = NC * NS
N_PAD = 10240
RPT = N_PAD // NS
E_PER_W = E // NW
CH = 80
N_CH = E_PER_W // CH


def _sc_deg(dst, zeros):
    mesh = plsc.VectorSubcoreMesh(core_axis_name="c", subcore_axis_name="s")

    @functools.partial(
        pl.kernel,
        out_type=jax.ShapeDtypeStruct((NC * N_PAD,), jnp.float32),
        mesh=mesh,
        scratch_types=[
            pltpu.VMEM_SHARED((N_PAD,), jnp.float32),
            pltpu.VMEM((CH,), jnp.int32),
            pltpu.VMEM((CH,), jnp.float32),
        ],
        compiler_params=pltpu.CompilerParams(use_tc_tiling_on_sc=False),
    )
    def k(dst_hbm, zero_hbm, deg_hbm, deg_sh, didx, ones):
        cid = lax.axis_index("c")
        sid = lax.axis_index("s")
        wid = cid * NS + sid
        r0 = sid * RPT
        pltpu.sync_copy(zero_hbm.at[pl.ds(r0, RPT)], deg_sh.at[pl.ds(r0, RPT)])
        for i in range(CH // 16):
            ones[pl.ds(i * 16, 16)] = jnp.full((16,), 1.0, jnp.float32)
        plsc.subcore_barrier()
        ebase = wid * E_PER_W

        @pl.loop(0, N_CH)
        def _(g):
            b = ebase + g * CH
            pltpu.sync_copy(dst_hbm.at[pl.ds(b, CH)], didx)
            pltpu.sync_copy(ones, deg_sh.at[didx], add=True)

        plsc.subcore_barrier()
        o0 = cid * N_PAD + r0
        pltpu.sync_copy(deg_sh.at[pl.ds(r0, RPT)], deg_hbm.at[pl.ds(o0, RPT)])

    return k(dst, zeros)


def _sc_agg(msg, src, dst, zeros, dw):
    mesh = plsc.VectorSubcoreMesh(core_axis_name="c", subcore_axis_name="s")

    @functools.partial(
        pl.kernel,
        out_type=jax.ShapeDtypeStruct((NC * N_PAD, dw), jnp.float32),
        mesh=mesh,
        scratch_types=[
            pltpu.VMEM_SHARED((N_PAD, dw), jnp.float32),
            pltpu.VMEM((CH,), jnp.int32),
            pltpu.VMEM((CH,), jnp.int32),
            pltpu.VMEM((CH, dw), jnp.float32),
            pltpu.SemaphoreType.DMA,
        ],
    )
    def k(msg_hbm, src_hbm, dst_hbm, zero_hbm, agg_hbm, acc_sh, sidx, didx,
          rows, sem):
        cid = lax.axis_index("c")
        sid = lax.axis_index("s")
        wid = cid * NS + sid
        r0 = sid * RPT
        pltpu.sync_copy(zero_hbm.at[pl.ds(r0, RPT)], acc_sh.at[pl.ds(r0, RPT)])
        plsc.subcore_barrier()
        ebase = wid * E_PER_W

        @pl.loop(0, N_CH)
        def _(g):
            b = ebase + g * CH
            pltpu.sync_copy(src_hbm.at[pl.ds(b, CH)], sidx)
            pltpu.sync_copy(dst_hbm.at[pl.ds(b, CH)], didx)
            pltpu.async_copy(msg_hbm.at[sidx], rows, sem).wait()
            pltpu.sync_copy(rows, acc_sh.at[didx], add=True)

        plsc.subcore_barrier()
        o0 = cid * N_PAD + r0
        pltpu.sync_copy(acc_sh.at[pl.ds(r0, RPT)], agg_hbm.at[pl.ds(o0, RPT)])

    return k(msg, src, dst, zeros)


def _sc_score(h2, src, dst):
    mesh = plsc.VectorSubcoreMesh(core_axis_name="c", subcore_axis_name="s")

    @functools.partial(
        pl.kernel,
        out_type=jax.ShapeDtypeStruct((E,), jnp.float32),
        mesh=mesh,
        scratch_types=[
            pltpu.VMEM((CH,), jnp.int32),
            pltpu.VMEM((CH,), jnp.int32),
            pltpu.VMEM((CH, D), jnp.float32),
            pltpu.VMEM((CH, D), jnp.float32),
            pltpu.VMEM((CH,), jnp.float32),
            pltpu.SemaphoreType.DMA,
            pltpu.SemaphoreType.DMA,
        ],
        compiler_params=pltpu.CompilerParams(needs_layout_passes=False),
    )
    def k(h_hbm, src_hbm, dst_hbm, out_hbm, sidx, didx, ubuf, vbuf, sbuf,
          sem_u, sem_v):
        cid = lax.axis_index("c")
        sid = lax.axis_index("s")
        wid = cid * NS + sid
        ebase = wid * E_PER_W
        lanes = lax.broadcasted_iota(jnp.int32, (16,), 0)

        @pl.loop(0, N_CH)
        def _(g):
            b = ebase + g * CH
            pltpu.sync_copy(src_hbm.at[pl.ds(b, CH)], sidx)
            pltpu.sync_copy(dst_hbm.at[pl.ds(b, CH)], didx)
            cu = pltpu.async_copy(h_hbm.at[sidx], ubuf, sem_u)
            cv = pltpu.async_copy(h_hbm.at[didx], vbuf, sem_v)
            cu.wait()
            cv.wait()

            @pl.loop(0, CH // 16)
            def _(t):
                sv = jnp.zeros((16,), jnp.float32)
                for l in range(16):
                    e = t * 16 + l
                    acc = ubuf[e, pl.ds(0, 16)] * vbuf[e, pl.ds(0, 16)]
                    for kk in range(1, 8):
                        acc = acc + (ubuf[e, pl.ds(kk * 16, 16)] *
                                     vbuf[e, pl.ds(kk * 16, 16)])
                    sv = jnp.where(lanes == l, jnp.sum(acc), sv)
                sbuf[pl.ds(t * 16, 16)] = sv

            pltpu.sync_copy(sbuf, out_hbm.at[pl.ds(b, CH)])

    return k(h2, src, dst)


def _tc_mm1(x, w_self, w_neigh):
    def body(x_ref, ws_ref, wn_ref, xs_ref, xw_ref):
        xv = x_ref[...]
        xs_ref[...] = jnp.dot(xv, ws_ref[...], preferred_element_type=jnp.float32)
        xw_ref[...] = jnp.dot(xv, wn_ref[...], preferred_element_type=jnp.float32)

    return pl.pallas_call(
        body,
        out_shape=[jax.ShapeDtypeStruct((N, D), jnp.float32)] * 2,
    )(x, w_self, w_neigh)


def _tc_hidden(xs, agg_a, agg_b, deg_a, deg_b, b1, w_self2, w_neigh2):
    def body(xs_ref, aa_ref, ab_ref, da_ref, db_ref, b_ref, ws_ref, wn_ref,
             hs_ref, hw_ref):
        inv = 1.0 / jnp.maximum(da_ref[...] + db_ref[...], 1.0)
        h = jnp.maximum(
            xs_ref[...] + (aa_ref[...] + ab_ref[...]) * inv + b_ref[...], 0.0)
        hs_ref[...] = jnp.dot(h, ws_ref[...], preferred_element_type=jnp.float32)
        hw_ref[...] = jnp.dot(h, wn_ref[...], preferred_element_type=jnp.float32)

    return pl.pallas_call(
        body,
        out_shape=[jax.ShapeDtypeStruct((N, D), jnp.float32)] * 2,
    )(xs, agg_a, agg_b, deg_a, deg_b, b1, w_self2, w_neigh2)


def _tc_final(hs, agg_a, agg_b, deg_a, deg_b, b2):
    def body(hs_ref, aa_ref, ab_ref, da_ref, db_ref, b_ref, out_ref):
        inv = 1.0 / jnp.maximum(da_ref[...] + db_ref[...], 1.0)
        out_ref[...] = (hs_ref[...] +
                        (aa_ref[...] + ab_ref[...]) * inv + b_ref[...])

    return pl.pallas_call(
        body,
        out_shape=jax.ShapeDtypeStruct((N, D), jnp.float32),
    )(hs, agg_a, agg_b, deg_a, deg_b, b2)


def kernel(x, edge_index, W_self1, W_neigh1, b1, W_self2, W_neigh2, b2):
    src = edge_index[0]
    dst = edge_index[1]
    z2 = jnp.zeros((N_PAD, D), jnp.float32)
    z1d = jnp.zeros((N_PAD,), jnp.float32)

    deg = _sc_deg(dst, z1d)
    deg_a, deg_b = deg[0:N, None], deg[N_PAD:N_PAD + N, None]

    xs1, xw1 = _tc_mm1(x, W_self1, W_neigh1)
    acc1 = _sc_agg(xw1, src, dst, z2, D)
    agg1a, agg1b = acc1[0:N], acc1[N_PAD:N_PAD + N]

    hs2, hw2 = _tc_hidden(xs1, agg1a, agg1b, deg_a, deg_b, b1[None, :],
                          W_self2, W_neigh2)
    acc2 = _sc_agg(hw2, src, dst, z2, D)
    h2 = _tc_final(hs2, acc2[0:N], acc2[N_PAD:N_PAD + N], deg_a, deg_b,
                   b2[None, :])

    score = _sc_score(h2, src, dst)
    return score[:, None]
